# baseline (speedup 1.0000x reference)
import functools

import jax
import jax.numpy as jnp
from jax import lax
from jax.experimental import pallas as pl
from jax.experimental.pallas import tpu as pltpu
from jax.experimental.pallas import tpu_sc as plsc


def _make_sc_embed(B, S, D, n_per, n_cores, n_sub, n_chunks):
    R = n_per // n_chunks
    n_slices = (S // n_per) // n_cores
    ld_rows = (n_slices * n_per) // n_sub
    mesh = plsc.VectorSubcoreMesh(core_axis_name="c", subcore_axis_name="s")

    @functools.partial(
        pl.kernel,
        mesh=mesh,
        out_type=jax.ShapeDtypeStruct((B, S, D), jnp.float32),
        scratch_types=[
            pltpu.VMEM((n_per,), jnp.int32),
            pltpu.VMEM((n_per, D), jnp.float32),
            pltpu.VMEM((n_per, D), jnp.float32),
            pltpu.VMEM_SHARED((n_slices, n_per, D), jnp.float32),
        ]
        + [pltpu.SemaphoreType.DMA] * (3 * n_chunks + 1),
    )
    def body(x_hbm, tok_hbm, pos_hbm, out_hbm, idx_v, tok_v, acc_v, pos_sh, *sems):
        gsems = sems[:n_chunks]
        psems = sems[n_chunks : 2 * n_chunks]
        osems = sems[2 * n_chunks : 3 * n_chunks]
        lsem = sems[3 * n_chunks]
        sid = lax.axis_index("s")
        cid = lax.axis_index("c")
        wid = sid * n_cores + cid
        base = wid * n_per
        b = base // S
        s0 = lax.rem(base, S)

        per_slice = n_sub // n_slices
        jl = sid // per_slice
        ro = lax.rem(sid, per_slice) * ld_rows
        gstart = pl.multiple_of((n_cores * jl + cid) * n_per + ro, ld_rows)


        pltpu.sync_copy(x_hbm.at[b, pl.ds(s0, n_per)], idx_v)
        gathers = []
        for i in range(0, n_chunks):
            sl = pl.ds(i * R, R)
            gathers.append(
                pltpu.async_copy(tok_hbm.at[idx_v.at[sl]], tok_v.at[sl], gsems[i])
            )

        j = lax.rem(sid, n_slices)
        poss = []
        for i in range(n_chunks):
            poss.append(
                pltpu.async_copy(
                    pos_hbm.at[pl.ds(s0 + i * R, R)],
                    acc_v.at[pl.ds(i * R, R)],
                    psems[i],
                )
            )

        outs = []
        for i in range(n_chunks):
            gathers[i].wait()
            poss[i].wait()

            def row_body(r, carry):
                for c in range(D // 16):
                    csl = pl.ds(c * 16, 16)
                    plsc.addupdate(acc_v.at[r, csl], tok_v[r, csl])
                return carry

            lax.fori_loop(i * R, (i + 1) * R, row_body, 0)
            outs.append(
                pltpu.async_copy(
                    acc_v.at[pl.ds(i * R, R)],
                    out_hbm.at[b, pl.ds(s0 + i * R, R)],
                    osems[i],
                )
            )
        for o in outs:
            o.wait()

    return body


def kernel(x, token_table, pos_table):
    B, S = x.shape
    V, D = token_table.shape
    N = B * S
    info = plsc.get_sparse_core_info()
    nw = info.num_cores * info.num_subcores
    n_per = N // nw
    fn = _make_sc_embed(
        B, S, D, n_per, info.num_cores, info.num_subcores, n_chunks=8
    )
    xi = x if x.dtype == jnp.int32 else x.astype(jnp.int32)
    return fn(xi, token_table, pos_table)


# speedup vs baseline: 1.0210x; 1.0210x over previous
import functools

import jax
import jax.numpy as jnp
from jax import lax
from jax.experimental import pallas as pl
from jax.experimental.pallas import tpu as pltpu
from jax.experimental.pallas import tpu_sc as plsc

_PAIR = 512
_N0 = 240
_CHUNK = 32


def _chunks_for(n):
    offs, out = 0, []
    while offs < n:
        ln = min(_CHUNK, n - offs)
        out.append((offs, ln))
        offs += ln
    return out


def _make_sc_embed(B, S, D, n_cores, n_sub):
    assert B * S == n_sub * _PAIR and S % _PAIR == 0
    n_slices = 4
    n1 = _PAIR - _N0
    mesh = plsc.VectorSubcoreMesh(core_axis_name="c", subcore_axis_name="s")

    plans = {0: (12, 80, 3), 1: (8, 136, 2)}
    max_chunks = len(_chunks_for(n1))

    @functools.partial(
        pl.kernel,
        mesh=mesh,
        out_type=jax.ShapeDtypeStruct((B, S, D), jnp.float32),
        scratch_types=[
            pltpu.VMEM((_PAIR,), jnp.int32),
            pltpu.VMEM((n1, D), jnp.float32),
            pltpu.VMEM((n1, D), jnp.float32),
            pltpu.VMEM_SHARED((n_slices, n1, D), jnp.float32),
        ]
        + [pltpu.SemaphoreType.DMA] * (3 * max_chunks + 1),
    )
    def body(x_hbm, tok_hbm, pos_hbm, out_hbm, idx_v, tok_v, acc_v, pos_sh, *sems):
        gsems = sems[:max_chunks]
        psems = sems[max_chunks : 2 * max_chunks]
        osems = sems[2 * max_chunks : 3 * max_chunks]
        lsem = sems[3 * max_chunks]
        sid = lax.axis_index("s")
        cid = lax.axis_index("c")
        b = sid // (S // _PAIR)
        p0 = lax.rem(sid, S // _PAIR) * _PAIR
        j = lax.rem(sid, n_slices)

        def run_core(my_cid):
            start = my_cid * _N0
            n_mine = n1 if my_cid else _N0
            chunks = _chunks_for(n_mine)
            n_loaders, piece, per_slice = plans[my_cid]

            def stage_refs():
                jload = sid // per_slice
                ro = lax.rem(sid, per_slice) * piece
                gstart = pl.multiple_of(jload * _PAIR + ro + start, 8)
                return (
                    pos_hbm.at[pl.ds(gstart, piece)],
                    pos_sh.at[jload, pl.ds(ro, piece)],
                )

            @pl.when(sid < n_loaders)
            def _():
                src, dst = stage_refs()
                pltpu.async_copy(src, dst, lsem)

            pltpu.sync_copy(x_hbm.at[b, pl.ds(p0, _PAIR)], idx_v)
            gathers = []
            for i, (off, ln) in enumerate(chunks):
                gathers.append(
                    pltpu.async_copy(
                        tok_hbm.at[idx_v.at[pl.ds(start + off, ln)]],
                        tok_v.at[pl.ds(off, ln)],
                        gsems[i],
                    )
                )

            @pl.when(sid < n_loaders)
            def _():
                src, dst = stage_refs()
                pltpu.make_async_copy(src, dst, lsem).wait()

            plsc.subcore_barrier()
            poss = []
            for i, (off, ln) in enumerate(chunks):
                poss.append(
                    pltpu.async_copy(
                        pos_sh.at[j, pl.ds(off, ln)],
                        acc_v.at[pl.ds(off, ln)],
                        psems[i],
                    )
                )

            outs = []
            for i, (off, ln) in enumerate(chunks):
                gathers[i].wait()
                poss[i].wait()

                def row_body(r, carry):
                    for c in range(D // 16):
                        csl = pl.ds(c * 16, 16)
                        plsc.addupdate(acc_v.at[r, csl], tok_v[r, csl])
                    return carry

                lax.fori_loop(off, off + ln, row_body, 0)
                outs.append(
                    pltpu.async_copy(
                        acc_v.at[pl.ds(off, ln)],
                        out_hbm.at[b, pl.ds(p0 + start + off, ln)],
                        osems[i],
                    )
                )
            for o in outs:
                o.wait()

        @pl.when(cid == 0)
        def _():
            run_core(0)

        @pl.when(cid == 1)
        def _():
            run_core(1)

    return body


def kernel(x, token_table, pos_table):
    B, S = x.shape
    V, D = token_table.shape
    info = plsc.get_sparse_core_info()
    fn = _make_sc_embed(B, S, D, info.num_cores, info.num_subcores)
    xi = x if x.dtype == jnp.int32 else x.astype(jnp.int32)
    return fn(xi, token_table, pos_table)


# speedup vs baseline: 1.0311x; 1.0098x over previous
import functools

import jax
import jax.numpy as jnp
from jax import lax
from jax.experimental import pallas as pl
from jax.experimental.pallas import tpu as pltpu
from jax.experimental.pallas import tpu_sc as plsc


def _make_sc_embed(B, S, D, n_per, n_cores, n_sub, n_chunks):
    R = n_per // n_chunks
    n_slices = (S // n_per) // n_cores
    ld_rows = (n_slices * n_per) // n_sub
    mesh = plsc.VectorSubcoreMesh(core_axis_name="c", subcore_axis_name="s")

    @functools.partial(
        pl.kernel,
        mesh=mesh,
        out_type=jax.ShapeDtypeStruct((B, S, D), jnp.float32),
        scratch_types=[
            pltpu.VMEM((n_per,), jnp.int32),
            pltpu.VMEM((n_per, D), jnp.float32),
            pltpu.VMEM((n_per, D), jnp.float32),
            pltpu.VMEM_SHARED((n_slices, n_per, D), jnp.float32),
        ]
        + [pltpu.SemaphoreType.DMA] * (3 * n_chunks + 1),
    )
    def body(x_hbm, tok_hbm, pos_hbm, out_hbm, idx_v, tok_v, acc_v, pos_sh, *sems):
        gsems = sems[:n_chunks]
        psems = sems[n_chunks : 2 * n_chunks]
        osems = sems[2 * n_chunks : 3 * n_chunks]
        lsem = sems[3 * n_chunks]
        sid = lax.axis_index("s")
        cid = lax.axis_index("c")
        wid = sid * n_cores + cid
        base = wid * n_per
        b = base // S
        s0 = lax.rem(base, S)

        per_slice = n_sub // n_slices
        jl = sid // per_slice
        ro = lax.rem(sid, per_slice) * ld_rows
        gstart = pl.multiple_of((n_cores * jl + cid) * n_per + ro, ld_rows)
        pload = pltpu.async_copy(
            pos_hbm.at[pl.ds(gstart, ld_rows)],
            pos_sh.at[jl, pl.ds(ro, ld_rows)],
            lsem,
        )

        pltpu.sync_copy(x_hbm.at[b, pl.ds(s0, n_per)], idx_v)
        pload.wait()
        plsc.subcore_barrier()
        j = lax.rem(sid, n_slices)
        poss = []
        for i in range(n_chunks):
            poss.append(
                pltpu.async_copy(
                    pos_sh.at[j, pl.ds(i * R, R)],
                    acc_v.at[pl.ds(i * R, R)],
                    psems[i],
                )
            )
        gathers = []
        for i in range(0, n_chunks):
            sl = pl.ds(i * R, R)
            gathers.append(
                pltpu.async_copy(tok_hbm.at[idx_v.at[sl]], tok_v.at[sl], gsems[i])
            )

        outs = []
        for i in range(n_chunks):
            gathers[i].wait()
            poss[i].wait()

            def row_body(r, carry):
                for c in range(D // 16):
                    csl = pl.ds(c * 16, 16)
                    plsc.addupdate(acc_v.at[r, csl], tok_v[r, csl])
                return carry

            lax.fori_loop(i * R, (i + 1) * R, row_body, 0)
            outs.append(
                pltpu.async_copy(
                    acc_v.at[pl.ds(i * R, R)],
                    out_hbm.at[b, pl.ds(s0 + i * R, R)],
                    osems[i],
                )
            )
        for o in outs:
            o.wait()

    return body


def kernel(x, token_table, pos_table):
    B, S = x.shape
    V, D = token_table.shape
    N = B * S
    info = plsc.get_sparse_core_info()
    nw = info.num_cores * info.num_subcores
    n_per = N // nw
    fn = _make_sc_embed(
        B, S, D, n_per, info.num_cores, info.num_subcores, n_chunks=8
    )
    xi = x if x.dtype == jnp.int32 else x.astype(jnp.int32)
    return fn(xi, token_table, pos_table)


# speedup vs baseline: 1.0552x; 1.0235x over previous
import functools

import jax
import jax.numpy as jnp
from jax import lax
from jax.experimental import pallas as pl
from jax.experimental.pallas import tpu as pltpu
from jax.experimental.pallas import tpu_sc as plsc


def _make_sc_embed(B, S, D, n_per, n_cores, n_sub, n_chunks):
    R = n_per // n_chunks
    n_slices = (S // n_per) // n_cores
    ld_rows = (n_slices * n_per) // n_sub
    mesh = plsc.VectorSubcoreMesh(core_axis_name="c", subcore_axis_name="s")

    @functools.partial(
        pl.kernel,
        mesh=mesh,
        out_type=jax.ShapeDtypeStruct((B, S, D), jnp.float32),
        scratch_types=[
            pltpu.VMEM((n_per,), jnp.int32),
            pltpu.VMEM((n_per, D), jnp.float32),
            pltpu.VMEM((n_per, D), jnp.float32),
            pltpu.VMEM_SHARED((n_slices, n_per, D), jnp.float32),
        ]
        + [pltpu.SemaphoreType.DMA] * (3 * n_chunks + 1),
    )
    def body(x_hbm, tok_hbm, pos_hbm, out_hbm, idx_v, tok_v, acc_v, pos_sh, *sems):
        gsems = sems[:n_chunks]
        psems = sems[n_chunks : 2 * n_chunks]
        osems = sems[2 * n_chunks : 3 * n_chunks]
        lsem = sems[3 * n_chunks]
        sid = lax.axis_index("s")
        cid = lax.axis_index("c")
        wid = sid * n_cores + cid
        base = wid * n_per
        b = base // S
        s0 = lax.rem(base, S)

        per_slice = n_sub // n_slices
        jl = sid // per_slice
        ro = lax.rem(sid, per_slice) * ld_rows
        gstart = pl.multiple_of((n_cores * jl + cid) * n_per + ro, ld_rows)
        pload = pltpu.async_copy(
            pos_hbm.at[pl.ds(gstart, ld_rows)],
            pos_sh.at[jl, pl.ds(ro, ld_rows)],
            lsem,
        )

        pltpu.sync_copy(x_hbm.at[b, pl.ds(s0, n_per)], idx_v)
        gathers = []
        for i in range(0, n_chunks):
            sl = pl.ds(i * R, R)
            gathers.append(
                pltpu.async_copy(tok_hbm.at[idx_v.at[sl]], tok_v.at[sl], gsems[i])
            )

        pload.wait()
        plsc.subcore_barrier()
        j = lax.rem(sid, n_slices)
        poss = []
        for i in range(n_chunks):
            poss.append(
                pltpu.async_copy(
                    pos_sh.at[j, pl.ds(i * R, R)],
                    acc_v.at[pl.ds(i * R, R)],
                    psems[i],
                )
            )

        outs = []
        for i in range(n_chunks):
            gathers[i].wait()
            poss[i].wait()

            def row_body(r, carry):
                for c in range(D // 16):
                    csl = pl.ds(c * 16, 16)
                    plsc.addupdate(acc_v.at[r, csl], tok_v[r, csl])
                return carry

            lax.fori_loop(i * R, (i + 1) * R, row_body, 0)
            outs.append(
                pltpu.async_copy(
                    acc_v.at[pl.ds(i * R, R)],
                    out_hbm.at[b, pl.ds(s0 + i * R, R)],
                    osems[i],
                )
            )
        for o in outs:
            o.wait()

    return body


def kernel(x, token_table, pos_table):
    B, S = x.shape
    V, D = token_table.shape
    N = B * S
    info = plsc.get_sparse_core_info()
    nw = info.num_cores * info.num_subcores
    n_per = N // nw
    fn = _make_sc_embed(
        B, S, D, n_per, info.num_cores, info.num_subcores, n_chunks=8
    )
    xi = x if x.dtype == jnp.int32 else x.astype(jnp.int32)
    return fn(xi, token_table, pos_table)
